# Initial kernel scaffold; baseline (speedup 1.0000x reference)
import functools

import jax
import jax.numpy as jnp
from jax import lax
from jax.experimental import pallas as pl
from jax.experimental.pallas import tpu as pltpu
from jax.experimental.pallas import tpu_sc as plsc

_B, _S, _D, _H, _E, _K = 2, 2048, 1024, 2048, 8, 2
_T = _B * _S
_N = _T * _K
_TILE = 512
_NT = _N // _TILE
_G = _NT + _E - 1
_GP = ((_G + 7) // 8) * 8
_HT = 1024
_NH = _H // _HT

_NC, _NS = 2, 16
_NW = _NC * _NS
_CT = 16

_NEG = jnp.float32(-1e30)


def _shift_cumsum(x, axis):
    n = x.shape[axis]
    s = 1
    while s < n:
        if axis == 0:
            pad = jnp.zeros((s,) + x.shape[1:], x.dtype)
            x = x + jnp.concatenate([pad, x[:-s]], axis=0)
        else:
            pad = jnp.zeros(x.shape[:-1] + (s,), x.dtype)
            x = x + jnp.concatenate([pad, x[..., :-s]], axis=-1)
        s *= 2
    return x


def _router_body(x_ref, wr_ref, br_ref, se_ref, rw_ref):
    logits = jnp.dot(x_ref[...], wr_ref[...],
                     preferred_element_type=jnp.float32,
                     precision=lax.Precision.HIGHEST)
    logits = logits[:, :_E] + br_ref[...]
    tt = logits.shape[0]
    iota_e = lax.broadcasted_iota(jnp.int32, (tt, _E), 1)
    m1 = jnp.max(logits, axis=1, keepdims=True)
    a1 = jnp.min(jnp.where(logits == m1, iota_e, _E), axis=1, keepdims=True)
    l2 = jnp.where(iota_e == a1, _NEG, logits)
    m2 = jnp.max(l2, axis=1, keepdims=True)
    a2 = jnp.min(jnp.where((l2 == m2) & (iota_e != a1), iota_e, _E),
                 axis=1, keepdims=True)
    w1 = 1.0 / (1.0 + jnp.exp(m2 - m1))
    w2 = 1.0 - w1
    se_ref[...] = jnp.concatenate([a1, a2], axis=1)
    rw_ref[...] = jnp.concatenate([w1, w2], axis=1)


def _router(x2d, wr_pad, br2d):
    tt = 1024
    return pl.pallas_call(
        _router_body,
        grid=(_T // tt,),
        in_specs=[
            pl.BlockSpec((tt, _D), lambda i: (i, 0)),
            pl.BlockSpec((_D, 128), lambda i: (0, 0)),
            pl.BlockSpec((1, _E), lambda i: (0, 0)),
        ],
        out_specs=[
            pl.BlockSpec((tt, _K), lambda i: (i, 0)),
            pl.BlockSpec((tt, _K), lambda i: (i, 0)),
        ],
        out_shape=[
            jax.ShapeDtypeStruct((_T, _K), jnp.int32),
            jax.ShapeDtypeStruct((_T, _K), jnp.float32),
        ],
    )(x2d, wr_pad, br2d)


def _plan_body(se_ref, pos0_ref, pos1_ref, usage_ref,
               tiles_ref, eps_ref, lo_ref, hi_ref):
    a1 = se_ref[:, 0:1]
    a2 = se_ref[:, 1:2]
    iota_e = lax.broadcasted_iota(jnp.int32, (_T, _E), 1)
    onehot0 = (iota_e == a1).astype(jnp.int32)
    onehot1 = (iota_e == a2).astype(jnp.int32)
    cnt = onehot0 + onehot1
    prefix_incl = _shift_cumsum(cnt, axis=0)
    prefix_excl = prefix_incl - cnt
    counts = prefix_incl[_T - 1:_T, :]
    off_incl = _shift_cumsum(counts, axis=1)
    off_excl = off_incl - counts
    slot = off_excl + prefix_excl
    pos0_ref[...] = jnp.sum(onehot0 * slot, axis=1, keepdims=True)
    pos1_ref[...] = jnp.sum(onehot1 * slot, axis=1, keepdims=True)
    usage_ref[...] = counts.astype(jnp.float32)

    first_tile = off_excl // _TILE
    last_tile = (off_incl - 1) // _TILE
    ntiles = jnp.where(counts > 0, last_tile - first_tile + 1, 0)
    steps_incl = _shift_cumsum(ntiles, axis=1)
    steps_excl = steps_incl - ntiles
    g_actual = steps_incl[:, _E - 1:_E]

    g_iota = lax.broadcasted_iota(jnp.int32, (_GP, _E), 0)
    e_iota = lax.broadcasted_iota(jnp.int32, (_GP, _E), 1)
    g_col = g_iota[:, 0:1]
    eps = jnp.sum((steps_incl <= g_iota).astype(jnp.int32),
                  axis=1, keepdims=True)
    eps = jnp.minimum(eps, _E - 1)
    valid = g_col < g_actual
    sel = (e_iota == eps).astype(jnp.int32)
    tiles = jnp.sum(sel * (first_tile + (g_col - steps_excl)),
                    axis=1, keepdims=True)
    tiles_ref[...] = jnp.where(valid, tiles, _NT - 1)
    eps_ref[...] = eps
    lo_ref[...] = jnp.where(
        valid, jnp.sum(sel * off_excl, axis=1, keepdims=True), 0)
    hi_ref[...] = jnp.where(
        valid, jnp.sum(sel * off_incl, axis=1, keepdims=True), 0)


def _plan(se):
    return pl.pallas_call(
        _plan_body,
        out_shape=[
            jax.ShapeDtypeStruct((_T, 1), jnp.int32),
            jax.ShapeDtypeStruct((_T, 1), jnp.int32),
            jax.ShapeDtypeStruct((1, _E), jnp.float32),
            jax.ShapeDtypeStruct((_GP, 1), jnp.int32),
            jax.ShapeDtypeStruct((_GP, 1), jnp.int32),
            jax.ShapeDtypeStruct((_GP, 1), jnp.int32),
            jax.ShapeDtypeStruct((_GP, 1), jnp.int32),
        ],
    )(se)


def _dispatch_body(x_hbm, pos0_hbm, pos1_hbm, xs_hbm,
                   idx0_v, idx1_v, rows_v, sem0, sem1):
    wid = lax.axis_index("s") * _NC + lax.axis_index("c")
    base = wid * (_T // _NW)

    def body(i, carry):
        tb = pl.multiple_of(base + i * _CT, _CT)
        pltpu.sync_copy(x_hbm.at[pl.ds(tb, _CT)], rows_v)
        pltpu.sync_copy(pos0_hbm.at[pl.ds(tb, _CT)], idx0_v)
        pltpu.sync_copy(pos1_hbm.at[pl.ds(tb, _CT)], idx1_v)
        cp0 = pltpu.async_copy(rows_v, xs_hbm.at[idx0_v], sem0)
        cp1 = pltpu.async_copy(rows_v, xs_hbm.at[idx1_v], sem1)
        cp0.wait()
        cp1.wait()
        return carry

    lax.fori_loop(0, _T // _NW // _CT, body, 0)


_dispatch = functools.partial(
    pl.kernel,
    out_type=jax.ShapeDtypeStruct((_N, _D), jnp.float32),
    mesh=plsc.VectorSubcoreMesh(core_axis_name="c", subcore_axis_name="s"),
    scratch_types=[
        pltpu.VMEM((_CT,), jnp.int32),
        pltpu.VMEM((_CT,), jnp.int32),
        pltpu.VMEM((_CT, _D), jnp.float32),
        pltpu.SemaphoreType.DMA,
        pltpu.SemaphoreType.DMA,
    ],
)(_dispatch_body)


def _mlp_body(tiles_s, eps_s, lo_s, hi_s,
              xs_ref, w1_ref, b1_ref, w2_ref, b2_ref, out_ref):
    g = pl.program_id(0)
    h = pl.program_id(1)
    x = xs_ref[...]
    hpre = jnp.dot(x, w1_ref[0], preferred_element_type=jnp.float32,
                   precision=lax.Precision.HIGHEST) + b1_ref[...]
    hact = jnp.maximum(hpre, 0.0)
    contrib = jnp.dot(hact, w2_ref[0], preferred_element_type=jnp.float32,
                      precision=lax.Precision.HIGHEST)
    contrib = contrib + jnp.where(h == 0, 1.0, 0.0) * b2_ref[...]
    r = tiles_s[g] * _TILE + lax.broadcasted_iota(jnp.int32, (_TILE, 1), 0)
    mask = ((r >= lo_s[g]) & (r < hi_s[g])).astype(jnp.float32)
    contrib = contrib * mask
    first = jnp.logical_and(
        jnp.logical_or(g == 0, tiles_s[g] != tiles_s[jnp.maximum(g - 1, 0)]),
        h == 0)

    @pl.when(first)
    def _():
        out_ref[...] = contrib

    @pl.when(jnp.logical_not(first))
    def _():
        out_ref[...] = out_ref[...] + contrib


def _mlp(tiles, eps, lo, hi, xs, w1, b1, w2, b2):
    grid_spec = pltpu.PrefetchScalarGridSpec(
        num_scalar_prefetch=4,
        grid=(_GP, _NH),
        in_specs=[
            pl.BlockSpec((_TILE, _D), lambda g, h, t, e, lo_, hi_: (t[g], 0)),
            pl.BlockSpec((1, _D, _HT), lambda g, h, t, e, lo_, hi_: (e[g], 0, h)),
            pl.BlockSpec((1, _HT), lambda g, h, t, e, lo_, hi_: (e[g], h)),
            pl.BlockSpec((1, _HT, _D), lambda g, h, t, e, lo_, hi_: (e[g], h, 0)),
            pl.BlockSpec((1, _D), lambda g, h, t, e, lo_, hi_: (e[g], 0)),
        ],
        out_specs=pl.BlockSpec((_TILE, _D), lambda g, h, t, e, lo_, hi_: (t[g], 0)),
    )
    return pl.pallas_call(
        _mlp_body,
        grid_spec=grid_spec,
        out_shape=jax.ShapeDtypeStruct((_N, _D), jnp.float32),
        compiler_params=pltpu.CompilerParams(
            dimension_semantics=("arbitrary", "arbitrary"),
            vmem_limit_bytes=100 * 1024 * 1024,
        ),
    )(tiles, eps, lo, hi, xs, w1, b1, w2, b2)


def _combine_body(s_hbm, pos0_hbm, pos1_hbm, w0_hbm, w1_hbm, out_hbm,
                  idx0_v, idx1_v, w0_v, w1_v, r0_v, r1_v, o_v, sem0, sem1):
    wid = lax.axis_index("s") * _NC + lax.axis_index("c")
    base = wid * (_T // _NW)

    def body(i, carry):
        tb = pl.multiple_of(base + i * _CT, _CT)
        pltpu.sync_copy(pos0_hbm.at[pl.ds(tb, _CT)], idx0_v)
        pltpu.sync_copy(pos1_hbm.at[pl.ds(tb, _CT)], idx1_v)
        pltpu.sync_copy(w0_hbm.at[pl.ds(tb, _CT)], w0_v)
        pltpu.sync_copy(w1_hbm.at[pl.ds(tb, _CT)], w1_v)
        cp0 = pltpu.async_copy(s_hbm.at[idx0_v], r0_v, sem0)
        cp1 = pltpu.async_copy(s_hbm.at[idx1_v], r1_v, sem1)
        cp0.wait()
        cp1.wait()
        for j in range(_CT):
            jsplat = jnp.full((16,), j, jnp.int32)
            w0s = plsc.load_gather(w0_v, [jsplat])
            w1s = plsc.load_gather(w1_v, [jsplat])

            def inner(c, carry2):
                sl = pl.ds(c * 16, 16)
                o_v[j, sl] = w0s * r0_v[j, sl] + w1s * r1_v[j, sl]
                return carry2

            lax.fori_loop(0, _D // 16, inner, 0)
        pltpu.sync_copy(o_v, out_hbm.at[pl.ds(tb, _CT)])
        return carry

    lax.fori_loop(0, _T // _NW // _CT, body, 0)


_combine = functools.partial(
    pl.kernel,
    out_type=jax.ShapeDtypeStruct((_T, _D), jnp.float32),
    mesh=plsc.VectorSubcoreMesh(core_axis_name="c", subcore_axis_name="s"),
    scratch_types=[
        pltpu.VMEM((_CT,), jnp.int32),
        pltpu.VMEM((_CT,), jnp.int32),
        pltpu.VMEM((_CT,), jnp.float32),
        pltpu.VMEM((_CT,), jnp.float32),
        pltpu.VMEM((_CT, _D), jnp.float32),
        pltpu.VMEM((_CT, _D), jnp.float32),
        pltpu.VMEM((_CT, _D), jnp.float32),
        pltpu.SemaphoreType.DMA,
        pltpu.SemaphoreType.DMA,
    ],
)(_combine_body)


def kernel(x, Wr, br, W1, b1, W2, b2, top_k):
    del top_k
    x2d = x.reshape(_T, _D)
    wr_pad = jnp.pad(Wr, ((0, 0), (0, 128 - _E)))
    br2d = br.reshape(1, _E)

    se, rw = _router(x2d, wr_pad, br2d)
    pos0, pos1, usage, tiles, eps, lo, hi = _plan(se)

    pos0 = pos0.reshape(_T)
    pos1 = pos1.reshape(_T)
    w0 = rw[:, 0].reshape(_T)
    w1 = rw[:, 1].reshape(_T)

    xs = _dispatch(x2d, pos0, pos1)
    s = _mlp(tiles.reshape(_GP), eps.reshape(_GP), lo.reshape(_GP),
             hi.reshape(_GP), xs, W1, b1, W2, b2)
    out = _combine(s, pos0, pos1, w0, w1)

    return out.reshape(_B, _S, _D), usage.reshape(_E)


# speedup vs baseline: 1.2689x; 1.2689x over previous
import functools

import jax
import jax.numpy as jnp
from jax import lax
from jax.experimental import pallas as pl
from jax.experimental.pallas import tpu as pltpu
from jax.experimental.pallas import tpu_sc as plsc

_B, _S, _D, _H, _E, _K = 2, 2048, 1024, 2048, 8, 2
_T = _B * _S
_N = _T * _K
_TILE = 512
_NT = _N // _TILE
_G = _NT + _E - 1
_GP = ((_G + 7) // 8) * 8
_HT = 1024
_NH = _H // _HT

_NC, _NS = 2, 16
_NW = _NC * _NS
_CT = 16

_NEG = -1e30


def _shift_cumsum(x, axis):
    n = x.shape[axis]
    s = 1
    while s < n:
        if axis == 0:
            pad = jnp.zeros((s,) + x.shape[1:], x.dtype)
            x = x + jnp.concatenate([pad, x[:-s]], axis=0)
        else:
            pad = jnp.zeros(x.shape[:-1] + (s,), x.dtype)
            x = x + jnp.concatenate([pad, x[..., :-s]], axis=-1)
        s *= 2
    return x


def _router_body(x_ref, wr_ref, br_ref, se_ref, rw_ref, w0x_ref, w1x_ref):
    logits = jnp.dot(x_ref[...], wr_ref[...],
                     preferred_element_type=jnp.float32,
                     precision=lax.Precision.DEFAULT)
    logits = logits[:, :_E] + br_ref[...]
    tt = logits.shape[0]
    iota_e = lax.broadcasted_iota(jnp.int32, (tt, _E), 1)
    m1 = jnp.max(logits, axis=1, keepdims=True)
    a1 = jnp.min(jnp.where(logits == m1, iota_e, _E), axis=1, keepdims=True)
    l2 = jnp.where(iota_e == a1, _NEG, logits)
    m2 = jnp.max(l2, axis=1, keepdims=True)
    a2 = jnp.min(jnp.where((l2 == m2) & (iota_e != a1), iota_e, _E),
                 axis=1, keepdims=True)
    w1 = 1.0 / (1.0 + jnp.exp(m2 - m1))
    w2 = 1.0 - w1
    se_ref[...] = jnp.concatenate([a1, a2], axis=1)
    rw_ref[...] = jnp.concatenate([w1, w2], axis=1)
    w0x_ref[...] = jnp.broadcast_to(w1, (tt, 16))
    w1x_ref[...] = jnp.broadcast_to(w2, (tt, 16))


def _router(x2d, wr_pad, br2d):
    tt = 1024
    return pl.pallas_call(
        _router_body,
        grid=(_T // tt,),
        in_specs=[
            pl.BlockSpec((tt, _D), lambda i: (i, 0)),
            pl.BlockSpec((_D, 128), lambda i: (0, 0)),
            pl.BlockSpec((1, _E), lambda i: (0, 0)),
        ],
        out_specs=[
            pl.BlockSpec((tt, _K), lambda i: (i, 0)),
            pl.BlockSpec((tt, _K), lambda i: (i, 0)),
            pl.BlockSpec((tt, 16), lambda i: (i, 0)),
            pl.BlockSpec((tt, 16), lambda i: (i, 0)),
        ],
        out_shape=[
            jax.ShapeDtypeStruct((_T, _K), jnp.int32),
            jax.ShapeDtypeStruct((_T, _K), jnp.float32),
            jax.ShapeDtypeStruct((_T, 16), jnp.float32),
            jax.ShapeDtypeStruct((_T, 16), jnp.float32),
        ],
    )(x2d, wr_pad, br2d)


def _plan_body(se_ref, pos0_ref, pos1_ref, usage_ref,
               tiles_ref, eps_ref, lo_ref, hi_ref):
    a1 = se_ref[:, 0:1]
    a2 = se_ref[:, 1:2]
    iota_e = lax.broadcasted_iota(jnp.int32, (_T, _E), 1)
    onehot0 = (iota_e == a1).astype(jnp.int32)
    onehot1 = (iota_e == a2).astype(jnp.int32)
    cnt = onehot0 + onehot1
    prefix_incl = _shift_cumsum(cnt, axis=0)
    prefix_excl = prefix_incl - cnt
    counts = prefix_incl[_T - 1:_T, :]
    off_incl = _shift_cumsum(counts, axis=1)
    off_excl = off_incl - counts
    slot = off_excl + prefix_excl
    pos0_ref[...] = jnp.sum(onehot0 * slot, axis=1, keepdims=True)
    pos1_ref[...] = jnp.sum(onehot1 * slot, axis=1, keepdims=True)
    usage_ref[...] = counts.astype(jnp.float32)

    first_tile = off_excl // _TILE
    last_tile = (off_incl - 1) // _TILE
    ntiles = jnp.where(counts > 0, last_tile - first_tile + 1, 0)
    steps_incl = _shift_cumsum(ntiles, axis=1)
    steps_excl = steps_incl - ntiles
    g_actual = steps_incl[:, _E - 1:_E]

    g_iota = lax.broadcasted_iota(jnp.int32, (_GP, _E), 0)
    e_iota = lax.broadcasted_iota(jnp.int32, (_GP, _E), 1)
    g_col = g_iota[:, 0:1]
    eps = jnp.sum((steps_incl <= g_iota).astype(jnp.int32),
                  axis=1, keepdims=True)
    eps = jnp.minimum(eps, _E - 1)
    valid = g_col < g_actual
    sel = (e_iota == eps).astype(jnp.int32)
    tiles = jnp.sum(sel * (first_tile + (g_col - steps_excl)),
                    axis=1, keepdims=True)
    tiles_ref[...] = jnp.where(valid, tiles, _NT - 1)
    eps_ref[...] = eps
    lo_ref[...] = jnp.where(
        valid, jnp.sum(sel * off_excl, axis=1, keepdims=True), 0)
    hi_ref[...] = jnp.where(
        valid, jnp.sum(sel * off_incl, axis=1, keepdims=True), 0)


def _plan(se):
    return pl.pallas_call(
        _plan_body,
        out_shape=[
            jax.ShapeDtypeStruct((_T, 1), jnp.int32),
            jax.ShapeDtypeStruct((_T, 1), jnp.int32),
            jax.ShapeDtypeStruct((1, _E), jnp.float32),
            jax.ShapeDtypeStruct((_GP, 1), jnp.int32),
            jax.ShapeDtypeStruct((_GP, 1), jnp.int32),
            jax.ShapeDtypeStruct((_GP, 1), jnp.int32),
            jax.ShapeDtypeStruct((_GP, 1), jnp.int32),
        ],
    )(se)


def _dispatch_body(x_hbm, pos0_hbm, pos1_hbm, xs_hbm,
                   idx0_v, idx1_v, rows_v, sem0, sem1):
    wid = lax.axis_index("s") * _NC + lax.axis_index("c")
    base = wid * (_T // _NW)

    def body(i, carry):
        tb = pl.multiple_of(base + i * _CT, _CT)
        pltpu.sync_copy(x_hbm.at[pl.ds(tb, _CT)], rows_v)
        pltpu.sync_copy(pos0_hbm.at[pl.ds(tb, _CT)], idx0_v)
        pltpu.sync_copy(pos1_hbm.at[pl.ds(tb, _CT)], idx1_v)
        cp0 = pltpu.async_copy(rows_v, xs_hbm.at[idx0_v], sem0)
        cp1 = pltpu.async_copy(rows_v, xs_hbm.at[idx1_v], sem1)
        cp0.wait()
        cp1.wait()
        return carry

    lax.fori_loop(0, _T // _NW // _CT, body, 0)


@functools.cache
def _get_dispatch():
    return functools.partial(
        pl.kernel,
        out_type=jax.ShapeDtypeStruct((_N, _D), jnp.float32),
        mesh=plsc.VectorSubcoreMesh(core_axis_name="c", subcore_axis_name="s",
                                    num_cores=_NC, num_subcores=_NS),
        scratch_types=[
            pltpu.VMEM((_CT,), jnp.int32),
            pltpu.VMEM((_CT,), jnp.int32),
            pltpu.VMEM((_CT, _D), jnp.float32),
            pltpu.SemaphoreType.DMA,
            pltpu.SemaphoreType.DMA,
        ],
    )(_dispatch_body)


def _mlp_body(tiles_s, eps_s, lo_s, hi_s,
              xs_ref, w1_ref, b1_ref, w2_ref, b2_ref, out_ref):
    g = pl.program_id(0)
    h = pl.program_id(1)
    x = xs_ref[...]
    hpre = jnp.dot(x, w1_ref[0], preferred_element_type=jnp.float32,
                   precision=lax.Precision.DEFAULT) + b1_ref[0]
    hact = jnp.maximum(hpre, 0.0)
    contrib = jnp.dot(hact, w2_ref[0], preferred_element_type=jnp.float32,
                      precision=lax.Precision.DEFAULT)
    contrib = contrib + jnp.where(h == 0, 1.0, 0.0) * b2_ref[0]
    r = tiles_s[g] * _TILE + lax.broadcasted_iota(jnp.int32, (_TILE, 1), 0)
    mask = ((r >= lo_s[g]) & (r < hi_s[g])).astype(jnp.float32)
    contrib = contrib * mask
    first = jnp.logical_and(
        jnp.logical_or(g == 0, tiles_s[g] != tiles_s[jnp.maximum(g - 1, 0)]),
        h == 0)

    @pl.when(first)
    def _():
        out_ref[...] = contrib

    @pl.when(jnp.logical_not(first))
    def _():
        out_ref[...] = out_ref[...] + contrib


def _mlp(tiles, eps, lo, hi, xs, w1, b1, w2, b2):
    grid_spec = pltpu.PrefetchScalarGridSpec(
        num_scalar_prefetch=4,
        grid=(_GP, _NH),
        in_specs=[
            pl.BlockSpec((_TILE, _D), lambda g, h, t, e, lo_, hi_: (t[g], 0)),
            pl.BlockSpec((1, _D, _HT), lambda g, h, t, e, lo_, hi_: (e[g], 0, h)),
            pl.BlockSpec((1, 1, _HT), lambda g, h, t, e, lo_, hi_: (e[g], 0, h)),
            pl.BlockSpec((1, _HT, _D), lambda g, h, t, e, lo_, hi_: (e[g], h, 0)),
            pl.BlockSpec((1, 1, _D), lambda g, h, t, e, lo_, hi_: (e[g], 0, 0)),
        ],
        out_specs=pl.BlockSpec((_TILE, _D), lambda g, h, t, e, lo_, hi_: (t[g], 0)),
    )
    return pl.pallas_call(
        _mlp_body,
        grid_spec=grid_spec,
        out_shape=jax.ShapeDtypeStruct((_N, _D), jnp.float32),
        compiler_params=pltpu.CompilerParams(
            dimension_semantics=("arbitrary", "arbitrary"),
            vmem_limit_bytes=100 * 1024 * 1024,
        ),
    )(tiles, eps, lo, hi, xs, w1, b1, w2, b2)


def _combine_body(s_hbm, pos0_hbm, pos1_hbm, w0x_hbm, w1x_hbm, out_hbm,
                  idx0_v, idx1_v, w0_v, w1_v, r0_v, r1_v, o_v, sem0, sem1):
    wid = lax.axis_index("s") * _NC + lax.axis_index("c")
    base = wid * (_T // _NW)

    def body(i, carry):
        tb = pl.multiple_of(base + i * _CT, _CT)
        pltpu.sync_copy(pos0_hbm.at[pl.ds(tb, _CT)], idx0_v)
        pltpu.sync_copy(pos1_hbm.at[pl.ds(tb, _CT)], idx1_v)
        pltpu.sync_copy(w0x_hbm.at[pl.ds(tb, _CT)], w0_v)
        pltpu.sync_copy(w1x_hbm.at[pl.ds(tb, _CT)], w1_v)
        cp0 = pltpu.async_copy(s_hbm.at[idx0_v], r0_v, sem0)
        cp1 = pltpu.async_copy(s_hbm.at[idx1_v], r1_v, sem1)
        cp0.wait()
        cp1.wait()
        for j in range(_CT):
            w0s = w0_v[j, :]
            w1s = w1_v[j, :]

            def inner(c, carry2):
                sl = pl.ds(c * 16, 16)
                o_v[j, sl] = w0s * r0_v[j, sl] + w1s * r1_v[j, sl]
                return carry2

            lax.fori_loop(0, _D // 16, inner, 0)
        pltpu.sync_copy(o_v, out_hbm.at[pl.ds(tb, _CT)])
        return carry

    lax.fori_loop(0, _T // _NW // _CT, body, 0)


@functools.cache
def _get_combine():
    return functools.partial(
        pl.kernel,
        out_type=jax.ShapeDtypeStruct((_T, _D), jnp.float32),
        mesh=plsc.VectorSubcoreMesh(core_axis_name="c", subcore_axis_name="s",
                                    num_cores=_NC, num_subcores=_NS),
        scratch_types=[
            pltpu.VMEM((_CT,), jnp.int32),
            pltpu.VMEM((_CT,), jnp.int32),
            pltpu.VMEM((_CT, 16), jnp.float32),
            pltpu.VMEM((_CT, 16), jnp.float32),
            pltpu.VMEM((_CT, _D), jnp.float32),
            pltpu.VMEM((_CT, _D), jnp.float32),
            pltpu.VMEM((_CT, _D), jnp.float32),
            pltpu.SemaphoreType.DMA,
            pltpu.SemaphoreType.DMA,
        ],
    )(_combine_body)


def kernel(x, Wr, br, W1, b1, W2, b2, top_k):
    del top_k
    x2d = x.reshape(_T, _D)
    wr_pad = jnp.pad(Wr, ((0, 0), (0, 128 - _E)))
    br2d = br.reshape(1, _E)

    se, rw, w0x, w1x = _router(x2d, wr_pad, br2d)
    del rw
    pos0, pos1, usage, tiles, eps, lo, hi = _plan(se)

    pos0 = pos0.reshape(_T)
    pos1 = pos1.reshape(_T)

    xs = _get_dispatch()(x2d, pos0, pos1)
    s = _mlp(tiles.reshape(_GP), eps.reshape(_GP), lo.reshape(_GP),
             hi.reshape(_GP), xs, W1, b1.reshape(_E, 1, _H),
             W2, b2.reshape(_E, 1, _D))
    out = _get_combine()(s, pos0, pos1, w0x, w1x)

    return out.reshape(_B, _S, _D), usage.reshape(_E)


# speedup vs baseline: 1.4838x; 1.1694x over previous
import functools

import jax
import jax.numpy as jnp
from jax import lax
from jax.experimental import pallas as pl
from jax.experimental.pallas import tpu as pltpu
from jax.experimental.pallas import tpu_sc as plsc

_B, _S, _D, _H, _E, _K = 2, 2048, 1024, 2048, 8, 2
_T = _B * _S
_N = _T * _K
_TILE = 256
_NT = _N // _TILE
_G = _NT + _E - 1
_GP = ((_G + 7) // 8) * 8

_NC, _NS = 2, 16
_NW = _NC * _NS
_CT = 32

_NEG = -1e30


def _shift_cumsum(x, axis):
    n = x.shape[axis]
    s = 1
    while s < n:
        if axis == 0:
            pad = jnp.zeros((s,) + x.shape[1:], x.dtype)
            x = x + jnp.concatenate([pad, x[:-s]], axis=0)
        else:
            pad = jnp.zeros(x.shape[:-1] + (s,), x.dtype)
            x = x + jnp.concatenate([pad, x[..., :-s]], axis=-1)
        s *= 2
    return x


def _router_body(x_ref, wr_ref, br_ref, se_ref, rw_ref, w0x_ref, w1x_ref):
    logits = jnp.dot(x_ref[...], wr_ref[...],
                     preferred_element_type=jnp.float32,
                     precision=lax.Precision.DEFAULT)
    logits = logits[:, :_E] + br_ref[...]
    tt = logits.shape[0]
    iota_e = lax.broadcasted_iota(jnp.int32, (tt, _E), 1)
    m1 = jnp.max(logits, axis=1, keepdims=True)
    a1 = jnp.min(jnp.where(logits == m1, iota_e, _E), axis=1, keepdims=True)
    l2 = jnp.where(iota_e == a1, _NEG, logits)
    m2 = jnp.max(l2, axis=1, keepdims=True)
    a2 = jnp.min(jnp.where((l2 == m2) & (iota_e != a1), iota_e, _E),
                 axis=1, keepdims=True)
    w1 = 1.0 / (1.0 + jnp.exp(m2 - m1))
    w2 = 1.0 - w1
    se_ref[...] = jnp.concatenate([a1, a2], axis=1)
    rw_ref[...] = jnp.concatenate([w1, w2], axis=1)
    w0x_ref[...] = jnp.broadcast_to(w1, (tt, 128))
    w1x_ref[...] = jnp.broadcast_to(w2, (tt, 128))


def _router(x2d, wr_pad, br2d):
    tt = 1024
    return pl.pallas_call(
        _router_body,
        grid=(_T // tt,),
        in_specs=[
            pl.BlockSpec((tt, _D), lambda i: (i, 0)),
            pl.BlockSpec((_D, 128), lambda i: (0, 0)),
            pl.BlockSpec((1, _E), lambda i: (0, 0)),
        ],
        out_specs=[
            pl.BlockSpec((tt, _K), lambda i: (i, 0)),
            pl.BlockSpec((tt, _K), lambda i: (i, 0)),
            pl.BlockSpec((tt, 128), lambda i: (i, 0)),
            pl.BlockSpec((tt, 128), lambda i: (i, 0)),
        ],
        out_shape=[
            jax.ShapeDtypeStruct((_T, _K), jnp.int32),
            jax.ShapeDtypeStruct((_T, _K), jnp.float32),
            jax.ShapeDtypeStruct((_T, 128), jnp.float32),
            jax.ShapeDtypeStruct((_T, 128), jnp.float32),
        ],
    )(x2d, wr_pad, br2d)


def _plan_body(se_ref, pos0_ref, pos1_ref, usage_ref,
               tiles_ref, eps_ref, lo_ref, hi_ref):
    a1 = se_ref[:, 0:1]
    a2 = se_ref[:, 1:2]
    iota_e = lax.broadcasted_iota(jnp.int32, (_T, _E), 1)
    onehot0 = (iota_e == a1).astype(jnp.int32)
    onehot1 = (iota_e == a2).astype(jnp.int32)
    cnt = onehot0 + onehot1
    prefix_incl = _shift_cumsum(cnt, axis=0)
    prefix_excl = prefix_incl - cnt
    counts = prefix_incl[_T - 1:_T, :]
    off_incl = _shift_cumsum(counts, axis=1)
    off_excl = off_incl - counts
    slot = off_excl + prefix_excl
    pos0_ref[...] = jnp.sum(onehot0 * slot, axis=1, keepdims=True)
    pos1_ref[...] = jnp.sum(onehot1 * slot, axis=1, keepdims=True)
    usage_ref[...] = counts.astype(jnp.float32)

    first_tile = off_excl // _TILE
    last_tile = (off_incl - 1) // _TILE
    ntiles = jnp.where(counts > 0, last_tile - first_tile + 1, 0)
    steps_incl = _shift_cumsum(ntiles, axis=1)
    steps_excl = steps_incl - ntiles
    g_actual = steps_incl[:, _E - 1:_E]

    g_iota = lax.broadcasted_iota(jnp.int32, (_GP, _E), 0)
    e_iota = lax.broadcasted_iota(jnp.int32, (_GP, _E), 1)
    g_col = g_iota[:, 0:1]
    eps = jnp.sum((steps_incl <= g_iota).astype(jnp.int32),
                  axis=1, keepdims=True)
    eps = jnp.minimum(eps, _E - 1)
    valid = g_col < g_actual
    sel = (e_iota == eps).astype(jnp.int32)
    tiles = jnp.sum(sel * (first_tile + (g_col - steps_excl)),
                    axis=1, keepdims=True)
    tiles_ref[...] = jnp.where(valid, tiles, _NT - 1)
    eps_ref[...] = eps
    lo_ref[...] = jnp.where(
        valid, jnp.sum(sel * off_excl, axis=1, keepdims=True), 0)
    hi_ref[...] = jnp.where(
        valid, jnp.sum(sel * off_incl, axis=1, keepdims=True), 0)


def _plan(se):
    return pl.pallas_call(
        _plan_body,
        out_shape=[
            jax.ShapeDtypeStruct((_T, 1), jnp.int32),
            jax.ShapeDtypeStruct((_T, 1), jnp.int32),
            jax.ShapeDtypeStruct((1, _E), jnp.float32),
            jax.ShapeDtypeStruct((_GP, 1), jnp.int32),
            jax.ShapeDtypeStruct((_GP, 1), jnp.int32),
            jax.ShapeDtypeStruct((_GP, 1), jnp.int32),
            jax.ShapeDtypeStruct((_GP, 1), jnp.int32),
        ],
    )(se)


def _dispatch_body(x_hbm, pos0_hbm, pos1_hbm, w0x_hbm, w1x_hbm,
                   xs_hbm, ws_hbm,
                   idx0_v, idx1_v, rows_v, w0r_v, w1r_v, sem0, sem1, sem2, sem3):
    wid = lax.axis_index("s") * _NC + lax.axis_index("c")
    base = wid * (_T // _NW)

    def body(i, carry):
        tb = pl.multiple_of(base + i * _CT, _CT)
        pltpu.sync_copy(x_hbm.at[pl.ds(tb, _CT)], rows_v)
        pltpu.sync_copy(pos0_hbm.at[pl.ds(tb, _CT)], idx0_v)
        pltpu.sync_copy(pos1_hbm.at[pl.ds(tb, _CT)], idx1_v)
        pltpu.sync_copy(w0x_hbm.at[pl.ds(tb, _CT)], w0r_v)
        pltpu.sync_copy(w1x_hbm.at[pl.ds(tb, _CT)], w1r_v)
        cp0 = pltpu.async_copy(rows_v, xs_hbm.at[idx0_v], sem0)
        cp1 = pltpu.async_copy(rows_v, xs_hbm.at[idx1_v], sem1)
        cp2 = pltpu.async_copy(w0r_v, ws_hbm.at[idx0_v], sem2)
        cp3 = pltpu.async_copy(w1r_v, ws_hbm.at[idx1_v], sem3)
        cp0.wait()
        cp1.wait()
        cp2.wait()
        cp3.wait()
        return carry

    lax.fori_loop(0, _T // _NW // _CT, body, 0)


@functools.cache
def _get_dispatch():
    return functools.partial(
        pl.kernel,
        out_type=(jax.ShapeDtypeStruct((_N, _D), jnp.float32),
                  jax.ShapeDtypeStruct((_N, 128), jnp.float32)),
        mesh=plsc.VectorSubcoreMesh(core_axis_name="c", subcore_axis_name="s",
                                    num_cores=_NC, num_subcores=_NS),
        scratch_types=[
            pltpu.VMEM((_CT,), jnp.int32),
            pltpu.VMEM((_CT,), jnp.int32),
            pltpu.VMEM((_CT, _D), jnp.float32),
            pltpu.VMEM((_CT, 128), jnp.float32),
            pltpu.VMEM((_CT, 128), jnp.float32),
            pltpu.SemaphoreType.DMA,
            pltpu.SemaphoreType.DMA,
            pltpu.SemaphoreType.DMA,
            pltpu.SemaphoreType.DMA,
        ],
    )(_dispatch_body)


def _mlp_body(tiles_s, eps_s, lo_s, hi_s,
              xs_ref, ws_ref, w1_ref, b1_ref, w2_ref, b2_ref, out_ref):
    g = pl.program_id(0)
    x = xs_ref[...]
    hpre = jnp.dot(x, w1_ref[0], preferred_element_type=jnp.float32,
                   precision=lax.Precision.DEFAULT) + b1_ref[0]
    hact = jnp.maximum(hpre, 0.0)
    contrib = jnp.dot(hact, w2_ref[0], preferred_element_type=jnp.float32,
                      precision=lax.Precision.DEFAULT)
    contrib = contrib + b2_ref[0]
    r = tiles_s[g] * _TILE + lax.broadcasted_iota(jnp.int32, (_TILE, 1), 0)
    mask = ((r >= lo_s[g]) & (r < hi_s[g])).astype(jnp.float32)
    contrib = contrib * (mask * ws_ref[:, 0:1])
    first = jnp.logical_or(g == 0, tiles_s[g] != tiles_s[jnp.maximum(g - 1, 0)])

    @pl.when(first)
    def _():
        out_ref[...] = contrib

    @pl.when(jnp.logical_not(first))
    def _():
        out_ref[...] = out_ref[...] + contrib


def _mlp(tiles, eps, lo, hi, xs, ws, w1, b1, w2, b2):
    grid_spec = pltpu.PrefetchScalarGridSpec(
        num_scalar_prefetch=4,
        grid=(_GP,),
        in_specs=[
            pl.BlockSpec((_TILE, _D), lambda g, t, e, lo_, hi_: (t[g], 0)),
            pl.BlockSpec((_TILE, 128), lambda g, t, e, lo_, hi_: (t[g], 0)),
            pl.BlockSpec((1, _D, _H), lambda g, t, e, lo_, hi_: (e[g], 0, 0)),
            pl.BlockSpec((1, 1, _H), lambda g, t, e, lo_, hi_: (e[g], 0, 0)),
            pl.BlockSpec((1, _H, _D), lambda g, t, e, lo_, hi_: (e[g], 0, 0)),
            pl.BlockSpec((1, 1, _D), lambda g, t, e, lo_, hi_: (e[g], 0, 0)),
        ],
        out_specs=pl.BlockSpec((_TILE, _D), lambda g, t, e, lo_, hi_: (t[g], 0)),
    )
    return pl.pallas_call(
        _mlp_body,
        grid_spec=grid_spec,
        out_shape=jax.ShapeDtypeStruct((_N, _D), jnp.float32),
        compiler_params=pltpu.CompilerParams(
            dimension_semantics=("arbitrary",),
            vmem_limit_bytes=100 * 1024 * 1024,
        ),
    )(tiles, eps, lo, hi, xs, ws, w1, b1, w2, b2)


def _combine_body(s_hbm, pos0_hbm, pos1_hbm, out_hbm,
                  idx0_v, idx1_v, r0_v, r1_v, sem0, sem1):
    wid = lax.axis_index("s") * _NC + lax.axis_index("c")
    base = wid * (_T // _NW)

    def body(i, carry):
        tb = pl.multiple_of(base + i * _CT, _CT)
        pltpu.sync_copy(pos0_hbm.at[pl.ds(tb, _CT)], idx0_v)
        pltpu.sync_copy(pos1_hbm.at[pl.ds(tb, _CT)], idx1_v)
        cp0 = pltpu.async_copy(s_hbm.at[idx0_v], r0_v, sem0)
        cp1 = pltpu.async_copy(s_hbm.at[idx1_v], r1_v, sem1)
        cp0.wait()
        cp1.wait()
        for j in range(_CT):

            def inner(c, carry2):
                for u in range(4):
                    sl = pl.ds(c * 64 + u * 16, 16)
                    r0_v[j, sl] = r0_v[j, sl] + r1_v[j, sl]
                return carry2

            lax.fori_loop(0, _D // 64, inner, 0)
        pltpu.sync_copy(r0_v, out_hbm.at[pl.ds(tb, _CT)])
        return carry

    lax.fori_loop(0, _T // _NW // _CT, body, 0)


@functools.cache
def _get_combine():
    return functools.partial(
        pl.kernel,
        out_type=jax.ShapeDtypeStruct((_T, _D), jnp.float32),
        mesh=plsc.VectorSubcoreMesh(core_axis_name="c", subcore_axis_name="s",
                                    num_cores=_NC, num_subcores=_NS),
        scratch_types=[
            pltpu.VMEM((_CT,), jnp.int32),
            pltpu.VMEM((_CT,), jnp.int32),
            pltpu.VMEM((_CT, _D), jnp.float32),
            pltpu.VMEM((_CT, _D), jnp.float32),
            pltpu.SemaphoreType.DMA,
            pltpu.SemaphoreType.DMA,
        ],
    )(_combine_body)


def kernel(x, Wr, br, W1, b1, W2, b2, top_k):
    del top_k
    x2d = x.reshape(_T, _D)
    wr_pad = jnp.pad(Wr, ((0, 0), (0, 128 - _E)))
    br2d = br.reshape(1, _E)

    se, rw, w0x, w1x = _router(x2d, wr_pad, br2d)
    del rw
    pos0, pos1, usage, tiles, eps, lo, hi = _plan(se)

    pos0 = pos0.reshape(_T)
    pos1 = pos1.reshape(_T)

    xs, ws = _get_dispatch()(x2d, pos0, pos1, w0x, w1x)
    s = _mlp(tiles.reshape(_GP), eps.reshape(_GP), lo.reshape(_GP),
             hi.reshape(_GP), xs, ws, W1, b1.reshape(_E, 1, _H),
             W2, b2.reshape(_E, 1, _D))
    out = _get_combine()(s, pos0, pos1)

    return out.reshape(_B, _S, _D), usage.reshape(_E)
